# baseline (speedup 1.0000x reference)
import jax
import jax.numpy as jnp
from jax.experimental import pallas as pl
from jax.experimental.pallas import tpu as pltpu

_N = 4
_C = 96
_H = 56
_PW = 58
_FLAT = _PW * _PW
_PADF = 3488
_KF = _C * 9
_HL = 3.0
_BND = 1.5


def _conv_kernel(x_ref, wT_ref, b_ref, out_ref, q_ref):
    @pl.when(pl.program_id(0) == 0)
    def _quantize():
        wT = wT_ref[...]
        step = jnp.max(jnp.abs(wT)) / _HL
        ws = wT / step
        f = jax.lax.broadcasted_iota(jnp.int32, (_KF, _KF), 0)
        g = jax.lax.broadcasted_iota(jnp.int32, (_KF, _KF), 1)
        pmat = jnp.where((f // 4) == (g // 4), 0.25, 0.0).astype(jnp.float32)
        cen = jnp.round(jnp.clip(
            jax.lax.dot(pmat, ws, precision=jax.lax.Precision.HIGHEST),
            -_BND, _BND))
        q = (jnp.round(jnp.clip(ws - cen, -_BND, _BND)) + cen) * step
        smat = jnp.where(g == (f % _C) * 9 + f // _C, 1.0, 0.0).astype(jnp.float32)
        q_ref[...] = jax.lax.dot(smat, q, precision=jax.lax.Precision.HIGHEST)

    x = x_ref[0]
    acc = jnp.zeros((_FLAT, _C), jnp.float32)
    for k in range(9):
        s = (k // 3) * _PW + (k % 3)
        a = x[s:s + _FLAT, :]
        wk = q_ref[k * _C:(k + 1) * _C, :]
        acc = acc + jax.lax.dot(a, wk, precision=jax.lax.Precision.HIGHEST)
    out_ref[0] = acc + b_ref[...]


@jax.jit
def kernel(x, weight, bias):
    n = x.shape[0]
    xt = jnp.transpose(x, (0, 2, 3, 1))
    xp = jnp.pad(xt, ((0, 0), (1, 1), (1, 1), (0, 0)))
    xf = xp.reshape(n, _FLAT, _C)
    xf = jnp.pad(xf, ((0, 0), (0, _PADF - _FLAT), (0, 0)))
    wT = jnp.transpose(weight.reshape(_C, _KF))
    b2 = bias.reshape(1, _C)

    out = pl.pallas_call(
        _conv_kernel,
        grid=(n,),
        in_specs=[
            pl.BlockSpec((1, _PADF, _C), lambda i: (i, 0, 0)),
            pl.BlockSpec((_KF, _C), lambda i: (0, 0)),
            pl.BlockSpec((1, _C), lambda i: (0, 0)),
        ],
        out_specs=pl.BlockSpec((1, _FLAT, _C), lambda i: (i, 0, 0)),
        out_shape=jax.ShapeDtypeStruct((n, _FLAT, _C), jnp.float32),
        scratch_shapes=[pltpu.VMEM((_KF, _C), jnp.float32)],
    )(xf, wT, b2)

    out = out.reshape(n, _PW, _PW, _C)[:, :_H, :_H, :]
    return jnp.transpose(out, (0, 3, 1, 2))
